# baseline (speedup 1.0000x reference)
import jax
import jax.numpy as jnp
from jax.experimental import pallas as pl

N = 10000
E = 320000
S = N // 2
MAX_NB = 64
R2 = 0.4 ** 2


NP_ = 10240
RWS = 8
CLS = NP_ // RWS


def _fps_kernel(px_ref, py_ref, pz_ref, out_ref):
    px = px_ref[...]
    py = py_ref[...]
    pz = pz_ref[...]
    iota = (jax.lax.broadcasted_iota(jnp.int32, (RWS, CLS), 0) * CLS
            + jax.lax.broadcasted_iota(jnp.int32, (RWS, CLS), 1))
    valid = iota < N
    qx = px[0, 0]
    qy = py[0, 0]
    qz = pz[0, 0]
    dx = px - qx
    dy = py - qy
    dz = pz - qz
    d0 = (dx * dx + dy * dy) + dz * dz
    d0 = jnp.where(valid, d0, -jnp.inf)
    out_ref[0:1, :] = jnp.zeros((1, 1), jnp.int32)

    def body(i, d):
        m = jnp.max(d)
        nxt = jnp.min(jnp.where(d == m, iota, jnp.int32(2 ** 30)))
        out_ref[pl.ds(i, 1), :] = jnp.reshape(nxt, (1, 1))
        sel = iota == nxt
        qx = jnp.sum(jnp.where(sel, px, 0.0))
        qy = jnp.sum(jnp.where(sel, py, 0.0))
        qz = jnp.sum(jnp.where(sel, pz, 0.0))
        ddx = px - qx
        ddy = py - qy
        ddz = pz - qz
        nd = (ddx * ddx + ddy * ddy) + ddz * ddz
        return jnp.minimum(d, nd)

    jax.lax.fori_loop(1, S, body, d0)


def _fps_pallas(pos):
    pp = jnp.pad(pos, ((0, NP_ - N), (0, 0)))
    pt = pp.T.reshape(3, RWS, CLS)
    out = pl.pallas_call(
        _fps_kernel,
        out_shape=jax.ShapeDtypeStruct((S, 1), jnp.int32),
    )(pt[0], pt[1], pt[2])
    return out.reshape(S)


def _gcn(h, src, dst, ew, W, b, n):
    m = (h @ W)[src] * ew[:, None]
    return jax.ops.segment_sum(m, dst, num_segments=n) + b


def _fps(pos, n_sample):
    p = jax.lax.stop_gradient(pos)
    d0 = jnp.sum((p - p[0]) ** 2, axis=1)
    def step(d, _):
        nxt = jnp.argmax(d)
        nd = jnp.sum((p - p[nxt]) ** 2, axis=1)
        return jnp.minimum(d, nd), nxt
    _, rest = jax.lax.scan(step, d0, None, length=n_sample - 1)
    return jnp.concatenate([jnp.zeros((1,), jnp.int32), rest.astype(jnp.int32)])


def _head_kernel(pooled_ref, wl_ref, bl_ref, logp_ref, pred_ref):
    out = jnp.dot(pooled_ref[...], wl_ref[...],
                  preferred_element_type=jnp.float32) + bl_ref[...]
    mx = jnp.max(out, axis=1, keepdims=True)
    sh = out - mx
    lse = jnp.log(jnp.sum(jnp.exp(sh), axis=1, keepdims=True))
    logp = sh - lse
    logp_ref[...] = logp
    p = jnp.exp(logp)
    pred_ref[...] = p / jnp.sum(p, axis=1, keepdims=True)


def kernel(norm, pos, x, batch, edge_index, W1, b1, W2, b2, W3, b3, Wl, bl):
    inp = jnp.concatenate([norm, pos, x], axis=1)
    src = edge_index[0]
    dst = edge_index[1]
    ew = jnp.ones((E,), jnp.float32)
    h = jax.nn.relu(_gcn(inp, src, dst, ew, W1, b1, N))
    h = jnp.concatenate([h, inp], axis=1)
    h = jax.nn.relu(_gcn(h, src, dst, ew, W2, b2, N))
    idx = _fps_pallas(pos)
    pos_q = pos[idx]
    d2 = (jnp.sum(pos_q * pos_q, axis=1)[:, None]
          + jnp.sum(pos * pos, axis=1)[None, :]
          - 2.0 * (pos_q @ pos.T))
    d2 = jnp.maximum(d2, 0.0)
    NCH = 16
    CW = 640
    d2p = jnp.pad(d2, ((0, 0), (0, NCH * CW - N)), constant_values=jnp.inf)
    d2c = d2p.reshape(S, NCH, CW)
    v1, i1 = jax.lax.top_k(-d2c, MAX_NB)
    g1 = i1 + (jnp.arange(NCH, dtype=jnp.int32) * CW)[None, :, None]
    v1f = v1.reshape(S, NCH * MAX_NB)
    g1f = g1.reshape(S, NCH * MAX_NB)
    negv, i2 = jax.lax.top_k(v1f, MAX_NB)
    nb = jnp.take_along_axis(g1f, i2, axis=1)
    valid = (-negv) <= R2
    x_j = h[nb]
    rel = pos[nb] - pos_q[:, None, :]
    msg = jnp.concatenate([x_j, rel], axis=-1)
    msg = jnp.where(valid[:, :, None], msg, -jnp.inf)
    pc = jnp.max(msg, axis=1)
    pc = jnp.where(jnp.isfinite(pc), pc, 0.0)
    mask = jnp.full((N,), -1, jnp.int32).at[idx].set(jnp.arange(S, dtype=jnp.int32))
    r = mask[src]
    c = mask[dst]
    ok = (r >= 0) & (c >= 0)
    src2 = jnp.where(ok, r, 0)
    dst2 = jnp.where(ok, c, 0)
    ew2 = jnp.where(ok, 1.0, 0.0).astype(jnp.float32)
    h3 = jnp.concatenate([pc, inp[idx]], axis=1)
    h3 = jax.nn.relu(_gcn(h3, src2, dst2, ew2, W3, b3, S))
    pooled = jax.ops.segment_max(h3, batch[idx], num_segments=1)
    logp, pred = pl.pallas_call(
        _head_kernel,
        out_shape=(
            jax.ShapeDtypeStruct((1, 10), jnp.float32),
            jax.ShapeDtypeStruct((1, 10), jnp.float32),
        ),
    )(pooled, Wl, bl.reshape(1, 10))
    return (logp, pred)


# speedup vs baseline: 1.9532x; 1.9532x over previous
import functools

import jax
import jax.numpy as jnp
from jax.experimental import pallas as pl

N = 10000
E = 320000
S = N // 2
MAX_NB = 64
R2 = 0.4 ** 2


NP_ = 10240
RWS = 8
CLS = NP_ // RWS


def _fps_kernel(px_ref, py_ref, pz_ref, out_ref):
    px = px_ref[...]
    py = py_ref[...]
    pz = pz_ref[...]
    iota = (jax.lax.broadcasted_iota(jnp.int32, (RWS, CLS), 0) * CLS
            + jax.lax.broadcasted_iota(jnp.int32, (RWS, CLS), 1))
    valid = iota < N
    qx = px[0, 0]
    qy = py[0, 0]
    qz = pz[0, 0]
    dx = px - qx
    dy = py - qy
    dz = pz - qz
    d0 = (dx * dx + dy * dy) + dz * dz
    d0 = jnp.where(valid, d0, -jnp.inf)
    out_ref[0:1, :] = jnp.zeros((1, 1), jnp.int32)

    def body(i, d):
        m = jnp.max(d)
        nxt = jnp.min(jnp.where(d == m, iota, jnp.int32(2 ** 30)))
        out_ref[pl.ds(i, 1), :] = jnp.reshape(nxt, (1, 1))
        sel = iota == nxt
        qx = jnp.sum(jnp.where(sel, px, 0.0))
        qy = jnp.sum(jnp.where(sel, py, 0.0))
        qz = jnp.sum(jnp.where(sel, pz, 0.0))
        ddx = px - qx
        ddy = py - qy
        ddz = pz - qz
        nd = (ddx * ddx + ddy * ddy) + ddz * ddz
        return jnp.minimum(d, nd)

    jax.lax.fori_loop(1, S, body, d0)


def _fps_pallas(pos):
    pp = jnp.pad(pos, ((0, NP_ - N), (0, 0)))
    pt = pp.T.reshape(3, RWS, CLS)
    out = pl.pallas_call(
        _fps_kernel,
        out_shape=jax.ShapeDtypeStruct((S, 1), jnp.int32),
    )(pt[0], pt[1], pt[2])
    return out.reshape(S)


KQ = 128
SQ = 5120


def _knn_kernel(qb_ref, p8_ref, nb_ref, dv_ref, *, npad, kq, k):
    qb = qb_ref[...]
    p8 = p8_ref[...]
    dotm = jnp.dot(qb, p8, preferred_element_type=jnp.float32)
    qq = qb[:, 4:5]
    pp = p8[3:4, :]
    d2 = (qq + pp) - 2.0 * dotm
    d2 = jnp.maximum(d2, 0.0)
    nid = jax.lax.broadcasted_iota(jnp.int32, (kq, npad), 1)
    lane = jax.lax.broadcasted_iota(jnp.int32, (kq, k), 1)
    BIGI = jnp.int32(2 ** 30)
    accid0 = jnp.zeros((kq, k), jnp.int32)
    accdv0 = jnp.zeros((kq, k), jnp.float32)

    def body(s, carry):
        d2c, accid, accdv = carry
        m = jnp.min(d2c, axis=1, keepdims=True)
        cand = jnp.where(d2c == m, nid, BIGI)
        sel = jnp.min(cand, axis=1, keepdims=True)
        d2c = jnp.where(cand == sel, jnp.inf, d2c)
        accid = jnp.where(lane == s, sel, accid)
        accdv = jnp.where(lane == s, m, accdv)
        return d2c, accid, accdv

    _, accid, accdv = jax.lax.fori_loop(0, k, body, (d2, accid0, accdv0))
    nb_ref[...] = accid
    dv_ref[...] = accdv


def _knn_pallas(pos_q, pos):
    qq = jnp.sum(pos_q * pos_q, axis=1)
    pp = jnp.sum(pos * pos, axis=1)
    qb = jnp.zeros((SQ, 8), jnp.float32)
    qb = qb.at[:S, 0:3].set(pos_q).at[:S, 4].set(qq)
    p8 = jnp.zeros((8, NP_), jnp.float32)
    p8 = p8.at[0:3, :N].set(pos.T).at[3, :N].set(pp)
    p8 = p8.at[3, N:].set(1e30)
    nb, dv = pl.pallas_call(
        functools.partial(_knn_kernel, npad=NP_, kq=KQ, k=MAX_NB),
        grid=(SQ // KQ,),
        in_specs=[
            pl.BlockSpec((KQ, 8), lambda b: (b, 0)),
            pl.BlockSpec((8, NP_), lambda b: (0, 0)),
        ],
        out_specs=[
            pl.BlockSpec((KQ, MAX_NB), lambda b: (b, 0)),
            pl.BlockSpec((KQ, MAX_NB), lambda b: (b, 0)),
        ],
        out_shape=[
            jax.ShapeDtypeStruct((SQ, MAX_NB), jnp.int32),
            jax.ShapeDtypeStruct((SQ, MAX_NB), jnp.float32),
        ],
    )(qb, p8)
    return nb[:S], dv[:S]


def _gcn(h, src, dst, ew, W, b, n):
    m = (h @ W)[src] * ew[:, None]
    return jax.ops.segment_sum(m, dst, num_segments=n) + b


def _fps(pos, n_sample):
    p = jax.lax.stop_gradient(pos)
    d0 = jnp.sum((p - p[0]) ** 2, axis=1)
    def step(d, _):
        nxt = jnp.argmax(d)
        nd = jnp.sum((p - p[nxt]) ** 2, axis=1)
        return jnp.minimum(d, nd), nxt
    _, rest = jax.lax.scan(step, d0, None, length=n_sample - 1)
    return jnp.concatenate([jnp.zeros((1,), jnp.int32), rest.astype(jnp.int32)])


def _head_kernel(pooled_ref, wl_ref, bl_ref, logp_ref, pred_ref):
    out = jnp.dot(pooled_ref[...], wl_ref[...],
                  preferred_element_type=jnp.float32) + bl_ref[...]
    mx = jnp.max(out, axis=1, keepdims=True)
    sh = out - mx
    lse = jnp.log(jnp.sum(jnp.exp(sh), axis=1, keepdims=True))
    logp = sh - lse
    logp_ref[...] = logp
    p = jnp.exp(logp)
    pred_ref[...] = p / jnp.sum(p, axis=1, keepdims=True)


def kernel(norm, pos, x, batch, edge_index, W1, b1, W2, b2, W3, b3, Wl, bl):
    inp = jnp.concatenate([norm, pos, x], axis=1)
    src = edge_index[0]
    dst = edge_index[1]
    ew = jnp.ones((E,), jnp.float32)
    h = jax.nn.relu(_gcn(inp, src, dst, ew, W1, b1, N))
    h = jnp.concatenate([h, inp], axis=1)
    h = jax.nn.relu(_gcn(h, src, dst, ew, W2, b2, N))
    idx = _fps_pallas(pos)
    pos_q = pos[idx]
    nb, dv = _knn_pallas(pos_q, pos)
    valid = dv <= R2
    x_j = h[nb]
    rel = pos[nb] - pos_q[:, None, :]
    msg = jnp.concatenate([x_j, rel], axis=-1)
    msg = jnp.where(valid[:, :, None], msg, -jnp.inf)
    pc = jnp.max(msg, axis=1)
    pc = jnp.where(jnp.isfinite(pc), pc, 0.0)
    mask = jnp.full((N,), -1, jnp.int32).at[idx].set(jnp.arange(S, dtype=jnp.int32))
    r = mask[src]
    c = mask[dst]
    ok = (r >= 0) & (c >= 0)
    src2 = jnp.where(ok, r, 0)
    dst2 = jnp.where(ok, c, 0)
    ew2 = jnp.where(ok, 1.0, 0.0).astype(jnp.float32)
    h3 = jnp.concatenate([pc, inp[idx]], axis=1)
    h3 = jax.nn.relu(_gcn(h3, src2, dst2, ew2, W3, b3, S))
    pooled = jax.ops.segment_max(h3, batch[idx], num_segments=1)
    logp, pred = pl.pallas_call(
        _head_kernel,
        out_shape=(
            jax.ShapeDtypeStruct((1, 10), jnp.float32),
            jax.ShapeDtypeStruct((1, 10), jnp.float32),
        ),
    )(pooled, Wl, bl.reshape(1, 10))
    return (logp, pred)
